# baseline (speedup 1.0000x reference)
import functools

import jax
import jax.numpy as jnp
from jax import lax
from jax.experimental import pallas as pl
from jax.experimental.pallas import tpu as pltpu
from jax.experimental.pallas import tpu_sc as plsc

VOCAB = 100000
EMBED = 64
WINDOW = 20
HIDDEN = 128

BV = 5000
NB = VOCAB // BV


_IDX_PAD = 32


@functools.cache
def _get_sc_gather():
    mesh = plsc.VectorSubcoreMesh(core_axis_name="c", subcore_axis_name="s")

    @functools.partial(
        pl.kernel,
        out_type=jax.ShapeDtypeStruct((WINDOW, EMBED), jnp.float32),
        mesh=mesh,
        scratch_types=[
            pltpu.VMEM((_IDX_PAD,), jnp.int32),
            pltpu.VMEM((WINDOW, EMBED), jnp.float32),
            pltpu.SemaphoreType.DMA,
        ],
        compiler_params=pltpu.CompilerParams(needs_layout_passes=False),
    )
    def _sc_gather(idx_hbm, emb_hbm, out_hbm, idx_v, sel_v, sem):
        c = lax.axis_index("c")
        s = lax.axis_index("s")

        @pl.when(jnp.logical_and(c == 0, s == 0))
        def _():
            pltpu.sync_copy(idx_hbm, idx_v.at[pl.ds(0, WINDOW)])
            lane = lax.iota(jnp.int32, 16)
            copies = []
            for r in range(WINDOW):
                chunk = idx_v[pl.ds((r // 16) * 16, 16)]
                xr = jnp.max(jnp.where(lane == (r % 16), chunk,
                                       jnp.zeros((16,), jnp.int32)))
                copies.append(pltpu.async_copy(
                    emb_hbm.at[pl.ds(xr, 1), :],
                    sel_v.at[pl.ds(r, 1), :],
                    sem,
                ))
            for cp in copies:
                cp.wait()
            pltpu.sync_copy(sel_v, out_hbm)

    return _sc_gather


_NT = (((1,), (1,)), ((), ()))

NSPLIT = 4
VQ = VOCAB // NSPLIT


def _mlp_body(g_ref, w1_ref, b1_ref, w2a_ref, w2b_ref, w2c_ref, w2d_ref,
              b2_ref, out_ref):
    z1 = lax.dot_general(g_ref[:], w1_ref[:], _NT,
                         preferred_element_type=jnp.float32)
    h = jnp.maximum(z1 + b1_ref[:], 0.0)

    zs = []
    for q, wq in enumerate((w2a_ref, w2b_ref, w2c_ref, w2d_ref)):
        z = (lax.dot_general(h, wq[0], _NT,
                             preferred_element_type=jnp.float32)
             + b2_ref[pl.ds(q, 1), :])
        zs.append(z)

    bms = [jnp.max(z, axis=1, keepdims=True) for z in zs]
    bm = jnp.maximum(jnp.maximum(bms[0], bms[1]),
                     jnp.maximum(bms[2], bms[3]))
    s = jnp.zeros((1, 1), jnp.float32)
    for z in zs:
        s = s + jnp.sum(jnp.exp(z - bm), axis=1, keepdims=True)
    norm = bm + jnp.log(s)
    for q, z in enumerate(zs):
        out_ref[pl.ds(q, 1), :] = z - norm


_w2_spec = [
    pl.BlockSpec((1, VQ, HIDDEN), (lambda q: (lambda j: (q, 0, 0)))(q))
    for q in range(NSPLIT)
]

_mlp_call = pl.pallas_call(
    _mlp_body,
    grid=(1,),
    in_specs=[
        pl.BlockSpec((1, WINDOW * EMBED), lambda j: (0, 0)),
        pl.BlockSpec((HIDDEN, WINDOW * EMBED), lambda j: (0, 0)),
        pl.BlockSpec((1, HIDDEN), lambda j: (0, 0)),
        *_w2_spec,
        pl.BlockSpec((NSPLIT, VQ), lambda j: (0, 0)),
    ],
    out_specs=pl.BlockSpec((NSPLIT, VQ), lambda j: (0, 0)),
    out_shape=jax.ShapeDtypeStruct((NSPLIT, VQ), jnp.float32),
)


def kernel(x, emb, W1, b1, W2, b2):
    g = _get_sc_gather()(x.astype(jnp.int32), emb)
    w2v = W2.reshape(NSPLIT, VQ, HIDDEN)
    out = _mlp_call(
        g.reshape(1, WINDOW * EMBED),
        W1,
        b1.reshape(1, HIDDEN),
        w2v, w2v, w2v, w2v,
        b2.reshape(NSPLIT, VQ),
    )
    return out.reshape(1, VOCAB)
